# baseline (speedup 1.0000x reference)
import functools

import jax
import jax.numpy as jnp
from jax import lax
from jax.experimental import pallas as pl
from jax.experimental.pallas import tpu as pltpu
from jax.experimental.pallas import tpu_sc as plsc

N = 10000
E = 320000
D = 128
C_OUT = 47
C_PAD = 64

NPAD = 10240
CHUNK = 128
NCHUNKS = 2560
EPAD = NCHUNKS * CHUNK
NW = 32
CHUNKS_PER_W = NCHUNKS // NW
ROWS_PER_TILE = NPAD // 16


def _make_sc_pass(width, with_deg, n0):
    mesh = plsc.VectorSubcoreMesh(core_axis_name="c", subcore_axis_name="s")
    nbuf = 2 if width > 64 else 4
    blk = 8
    n1 = 2 * CHUNKS_PER_W - n0
    assert n0 % blk == 0 and n1 % blk == 0
    out_type = [jax.ShapeDtypeStruct((2, NPAD, width), jnp.float32)]
    scratch = [
        pltpu.VMEM_SHARED((NPAD, width), jnp.float32),
        pltpu.VMEM((blk, CHUNK), jnp.int32),
        pltpu.VMEM((blk, CHUNK), jnp.int32),
    ] + [pltpu.VMEM((CHUNK, width), jnp.float32) for _ in range(nbuf)] \
      + [pltpu.SemaphoreType.DMA for _ in range(nbuf)]
    if with_deg:
        out_type.append(jax.ShapeDtypeStruct((2, NPAD, 16), jnp.float32))
        scratch += [
            pltpu.VMEM_SHARED((NPAD, 16), jnp.float32),
            pltpu.VMEM((CHUNK, 16), jnp.float32),
        ]

    def body(*refs):
        if with_deg:
            (table, src2d, dst2d, zero_w, zero_d, ones_hbm, outp, degp,
             agg_sp, sidx, didx, *rest) = refs
            msgs, sems = rest[:nbuf], rest[nbuf:2 * nbuf]
            deg_sp, ones_v = rest[2 * nbuf:]
        else:
            (table, src2d, dst2d, zero_w, outp,
             agg_sp, sidx, didx, *rest) = refs
            msgs, sems = rest[:nbuf], rest[nbuf:2 * nbuf]
        cid = lax.axis_index("c")
        sid = lax.axis_index("s")
        base = jnp.where(cid == 0, sid * n0, 16 * n0 + sid * n1)
        nblk_self = jnp.where(cid == 0, n0 // blk, n1 // blk)
        wslot = cid * 16 + sid
        row0 = sid * ROWS_PER_TILE
        rows = pl.ds(row0, ROWS_PER_TILE)

        pltpu.sync_copy(zero_w.at[wslot], agg_sp.at[rows])
        if with_deg:
            pltpu.sync_copy(zero_d.at[wslot], deg_sp.at[rows])
            pltpu.sync_copy(ones_hbm.at[wslot], ones_v)
        plsc.subcore_barrier()

        @pl.loop(0, nblk_self)
        def _(k):
            c0 = base + k * blk
            pltpu.sync_copy(src2d.at[pl.ds(c0, blk)], sidx)
            pltpu.sync_copy(dst2d.at[pl.ds(c0, blk)], didx)
            for b in range(nbuf):
                pltpu.async_copy(table.at[sidx.at[b]], msgs[b], sems[b])
            for j in range(blk):
                m = j % nbuf
                pltpu.make_async_copy(table.at[sidx.at[j]], msgs[m],
                                      sems[m]).wait()
                pltpu.sync_copy(msgs[m], agg_sp.at[didx.at[j]], add=True)
                if with_deg:
                    pltpu.sync_copy(ones_v, deg_sp.at[didx.at[j]], add=True)
                if j + nbuf < blk:
                    pltpu.async_copy(table.at[sidx.at[j + nbuf]], msgs[m],
                                     sems[m])

        plsc.subcore_barrier()
        pltpu.sync_copy(agg_sp.at[rows], outp.at[cid, rows])
        if with_deg:
            pltpu.sync_copy(deg_sp.at[rows], degp.at[cid, rows])

    return pl.kernel(body, out_type=out_type, mesh=mesh,
                     scratch_types=scratch,
                     compiler_params=pltpu.CompilerParams(
                         use_tc_tiling_on_sc=False))


def _dot(a, b):
    return lax.dot_general(a, b, (((1,), (0,)), ((), ())),
                           precision=lax.Precision.HIGHEST,
                           preferred_element_type=jnp.float32)


def _tc_layer_a_body(x_ref, p_ref, degp_ref, ws0_ref, wn0_ref, ws1_ref,
                     wn1_ref, b0_ref, b1_ref, z1_ref, s_ref):
    deg = jnp.maximum(degp_ref[0, :, 0:1] + degp_ref[1, :, 0:1], 1.0)
    m = (p_ref[0] + p_ref[1]) / deg
    h1 = jnp.maximum(
        _dot(x_ref[...], ws0_ref[...]) + _dot(m, wn0_ref[...]) + b0_ref[...],
        0.0)
    z1_ref[...] = _dot(h1, wn1_ref[...])
    s_ref[...] = _dot(h1, ws1_ref[...]) + b1_ref[...]


def _tc_layer_b_body(q_ref, degp_ref, s_ref, out_ref):
    deg = jnp.maximum(degp_ref[0, :, 0:1] + degp_ref[1, :, 0:1], 1.0)
    out_ref[...] = s_ref[...] + (q_ref[0] + q_ref[1]) / deg


_TC_R = 1000


def _tc_layer_a(x, p, degp, ws0, wn0, ws1p, wn1p, b0, b1p):
    grid = (N // _TC_R,)
    return pl.pallas_call(
        _tc_layer_a_body,
        grid=grid,
        in_specs=[
            pl.BlockSpec((_TC_R, D), lambda i: (i, 0)),
            pl.BlockSpec((2, _TC_R, D), lambda i: (0, i, 0)),
            pl.BlockSpec((2, _TC_R, 16), lambda i: (0, i, 0)),
            pl.BlockSpec((D, D), lambda i: (0, 0)),
            pl.BlockSpec((D, D), lambda i: (0, 0)),
            pl.BlockSpec((D, C_PAD), lambda i: (0, 0)),
            pl.BlockSpec((D, C_PAD), lambda i: (0, 0)),
            pl.BlockSpec((1, D), lambda i: (0, 0)),
            pl.BlockSpec((1, C_PAD), lambda i: (0, 0)),
        ],
        out_specs=[
            pl.BlockSpec((_TC_R, C_PAD), lambda i: (i, 0)),
            pl.BlockSpec((_TC_R, C_PAD), lambda i: (i, 0)),
        ],
        out_shape=[
            jax.ShapeDtypeStruct((N, C_PAD), jnp.float32),
            jax.ShapeDtypeStruct((N, C_PAD), jnp.float32),
        ],
    )(x, p, degp, ws0, wn0, ws1p, wn1p, b0, b1p)


def _tc_layer_b(q, degp, s):
    grid = (N // _TC_R,)
    return pl.pallas_call(
        _tc_layer_b_body,
        grid=grid,
        in_specs=[
            pl.BlockSpec((2, _TC_R, C_PAD), lambda i: (0, i, 0)),
            pl.BlockSpec((2, _TC_R, 16), lambda i: (0, i, 0)),
            pl.BlockSpec((_TC_R, C_PAD), lambda i: (i, 0)),
        ],
        out_specs=pl.BlockSpec((_TC_R, C_PAD), lambda i: (i, 0)),
        out_shape=jax.ShapeDtypeStruct((N, C_PAD), jnp.float32),
    )(q, degp, s)


def kernel(x, edge_index, W_self_0, W_neigh_0, b_0, W_self_1, W_neigh_1, b_1):
    src = edge_index[0].astype(jnp.int32)
    dst = edge_index[1].astype(jnp.int32)
    npad_e = EPAD - E
    src2d = jnp.concatenate(
        [src, jnp.zeros((npad_e,), jnp.int32)]).reshape(NCHUNKS, CHUNK)
    dst2d = jnp.concatenate(
        [dst, jnp.full((npad_e,), N, jnp.int32)]).reshape(NCHUNKS, CHUNK)

    zero_w = jnp.zeros((32, ROWS_PER_TILE, D), jnp.float32)
    zero_c = jnp.zeros((32, ROWS_PER_TILE, C_PAD), jnp.float32)
    zero_d = jnp.zeros((32, ROWS_PER_TILE, 16), jnp.float32)
    ones = jnp.ones((32, CHUNK, 16), jnp.float32)

    p, degp = _make_sc_pass(D, True, 128)(x, src2d, dst2d, zero_w, zero_d,
                                          ones)

    ws1p = jnp.pad(W_self_1, ((0, 0), (0, C_PAD - C_OUT)))
    wn1p = jnp.pad(W_neigh_1, ((0, 0), (0, C_PAD - C_OUT)))
    b1p = jnp.pad(b_1, (0, C_PAD - C_OUT)).reshape(1, C_PAD)
    b0r = b_0.reshape(1, D)

    z1, s = _tc_layer_a(x, p, degp, W_self_0, W_neigh_0, ws1p, wn1p, b0r, b1p)

    q = _make_sc_pass(C_PAD, False, 136)(z1, src2d, dst2d, zero_c)
    if isinstance(q, (list, tuple)):
        q = q[0]

    out = _tc_layer_b(q, degp, s)
    return out[:, :C_OUT]


# speedup vs baseline: 1.0004x; 1.0004x over previous
import functools

import jax
import jax.numpy as jnp
from jax import lax
from jax.experimental import pallas as pl
from jax.experimental.pallas import tpu as pltpu
from jax.experimental.pallas import tpu_sc as plsc

N = 10000
E = 320000
D = 128
C_OUT = 47
C_PAD = 64

NPAD = 10240
CHUNK = 128
NCHUNKS = 2560
EPAD = NCHUNKS * CHUNK
NW = 32
CHUNKS_PER_W = NCHUNKS // NW
ROWS_PER_TILE = NPAD // 16


def _make_sc_pass(width, with_deg, n0):
    mesh = plsc.VectorSubcoreMesh(core_axis_name="c", subcore_axis_name="s")
    nbuf = 2 if width > 64 else 4
    blk = 8
    n1 = 2 * CHUNKS_PER_W - n0
    assert n0 % blk == 0 and n1 % blk == 0
    out_type = [jax.ShapeDtypeStruct((2, NPAD, width), jnp.float32)]
    scratch = [
        pltpu.VMEM_SHARED((NPAD, width), jnp.float32),
        pltpu.VMEM((blk, CHUNK), jnp.int32),
        pltpu.VMEM((blk, CHUNK), jnp.int32),
    ] + [pltpu.VMEM((CHUNK, width), jnp.float32) for _ in range(nbuf)] \
      + [pltpu.SemaphoreType.DMA for _ in range(nbuf)]
    if with_deg:
        out_type.append(jax.ShapeDtypeStruct((2, NPAD, 16), jnp.float32))
        scratch += [
            pltpu.VMEM_SHARED((NPAD, 16), jnp.float32),
            pltpu.VMEM((CHUNK, 16), jnp.float32),
        ]

    def body(*refs):
        if with_deg:
            (table, src2d, dst2d, zero_w, zero_d, ones_hbm, outp, degp,
             agg_sp, sidx, didx, *rest) = refs
            msgs, sems = rest[:nbuf], rest[nbuf:2 * nbuf]
            deg_sp, ones_v = rest[2 * nbuf:]
        else:
            (table, src2d, dst2d, zero_w, outp,
             agg_sp, sidx, didx, *rest) = refs
            msgs, sems = rest[:nbuf], rest[nbuf:2 * nbuf]
        cid = lax.axis_index("c")
        sid = lax.axis_index("s")
        base = jnp.where(cid == 0, sid * n0, 16 * n0 + sid * n1)
        nblk_self = jnp.where(cid == 0, n0 // blk, n1 // blk)
        wslot = cid * 16 + sid
        row0 = sid * ROWS_PER_TILE
        rows = pl.ds(row0, ROWS_PER_TILE)

        with jax.named_scope("sc_zero_fill"):
            pltpu.sync_copy(zero_w.at[wslot], agg_sp.at[rows])
            if with_deg:
                pltpu.sync_copy(zero_d.at[wslot], deg_sp.at[rows])
                pltpu.sync_copy(ones_hbm.at[wslot], ones_v)
            plsc.subcore_barrier()

        with jax.named_scope("sc_edge_loop"):
            @pl.loop(0, nblk_self)
            def _(k):
                c0 = base + k * blk
                pltpu.sync_copy(src2d.at[pl.ds(c0, blk)], sidx)
                pltpu.sync_copy(dst2d.at[pl.ds(c0, blk)], didx)
                for b in range(nbuf):
                    pltpu.async_copy(table.at[sidx.at[b]], msgs[b], sems[b])
                for j in range(blk):
                    m = j % nbuf
                    pltpu.make_async_copy(table.at[sidx.at[j]], msgs[m],
                                          sems[m]).wait()
                    pltpu.sync_copy(msgs[m], agg_sp.at[didx.at[j]], add=True)
                    if with_deg:
                        pltpu.sync_copy(ones_v, deg_sp.at[didx.at[j]],
                                        add=True)
                    if j + nbuf < blk:
                        pltpu.async_copy(table.at[sidx.at[j + nbuf]],
                                         msgs[m], sems[m])

        with jax.named_scope("sc_writeback"):
            plsc.subcore_barrier()
            pltpu.sync_copy(agg_sp.at[rows], outp.at[cid, rows])
            if with_deg:
                pltpu.sync_copy(deg_sp.at[rows], degp.at[cid, rows])

    return pl.kernel(body, out_type=out_type, mesh=mesh,
                     scratch_types=scratch,
                     compiler_params=pltpu.CompilerParams(
                         use_tc_tiling_on_sc=False))


def _dot(a, b):
    return lax.dot_general(a, b, (((1,), (0,)), ((), ())),
                           precision=lax.Precision.HIGHEST,
                           preferred_element_type=jnp.float32)


def _tc_layer_a_body(x_ref, p_ref, degp_ref, ws0_ref, wn0_ref, ws1_ref,
                     wn1_ref, b0_ref, b1_ref, z1_ref, s_ref):
    deg = jnp.maximum(degp_ref[0, :, 0:1] + degp_ref[1, :, 0:1], 1.0)
    m = (p_ref[0] + p_ref[1]) / deg
    h1 = jnp.maximum(
        _dot(x_ref[...], ws0_ref[...]) + _dot(m, wn0_ref[...]) + b0_ref[...],
        0.0)
    z1_ref[...] = _dot(h1, wn1_ref[...])
    s_ref[...] = _dot(h1, ws1_ref[...]) + b1_ref[...]


def _tc_layer_b_body(q_ref, degp_ref, s_ref, out_ref):
    deg = jnp.maximum(degp_ref[0, :, 0:1] + degp_ref[1, :, 0:1], 1.0)
    out_ref[...] = s_ref[...] + (q_ref[0] + q_ref[1]) / deg


_TC_R = 1000


def _tc_layer_a(x, p, degp, ws0, wn0, ws1p, wn1p, b0, b1p):
    grid = (N // _TC_R,)
    return pl.pallas_call(
        _tc_layer_a_body,
        grid=grid,
        in_specs=[
            pl.BlockSpec((_TC_R, D), lambda i: (i, 0)),
            pl.BlockSpec((2, _TC_R, D), lambda i: (0, i, 0)),
            pl.BlockSpec((2, _TC_R, 16), lambda i: (0, i, 0)),
            pl.BlockSpec((D, D), lambda i: (0, 0)),
            pl.BlockSpec((D, D), lambda i: (0, 0)),
            pl.BlockSpec((D, C_PAD), lambda i: (0, 0)),
            pl.BlockSpec((D, C_PAD), lambda i: (0, 0)),
            pl.BlockSpec((1, D), lambda i: (0, 0)),
            pl.BlockSpec((1, C_PAD), lambda i: (0, 0)),
        ],
        out_specs=[
            pl.BlockSpec((_TC_R, C_PAD), lambda i: (i, 0)),
            pl.BlockSpec((_TC_R, C_PAD), lambda i: (i, 0)),
        ],
        out_shape=[
            jax.ShapeDtypeStruct((N, C_PAD), jnp.float32),
            jax.ShapeDtypeStruct((N, C_PAD), jnp.float32),
        ],
    )(x, p, degp, ws0, wn0, ws1p, wn1p, b0, b1p)


def _tc_layer_b(q, degp, s):
    grid = (N // _TC_R,)
    return pl.pallas_call(
        _tc_layer_b_body,
        grid=grid,
        in_specs=[
            pl.BlockSpec((2, _TC_R, C_PAD), lambda i: (0, i, 0)),
            pl.BlockSpec((2, _TC_R, 16), lambda i: (0, i, 0)),
            pl.BlockSpec((_TC_R, C_PAD), lambda i: (i, 0)),
        ],
        out_specs=pl.BlockSpec((_TC_R, C_PAD), lambda i: (i, 0)),
        out_shape=jax.ShapeDtypeStruct((N, C_PAD), jnp.float32),
    )(q, degp, s)


def kernel(x, edge_index, W_self_0, W_neigh_0, b_0, W_self_1, W_neigh_1, b_1):
    src = edge_index[0].astype(jnp.int32)
    dst = edge_index[1].astype(jnp.int32)
    npad_e = EPAD - E
    src2d = jnp.concatenate(
        [src, jnp.zeros((npad_e,), jnp.int32)]).reshape(NCHUNKS, CHUNK)
    dst2d = jnp.concatenate(
        [dst, jnp.full((npad_e,), N, jnp.int32)]).reshape(NCHUNKS, CHUNK)

    zero_w = jnp.zeros((32, ROWS_PER_TILE, D), jnp.float32)
    zero_c = jnp.zeros((32, ROWS_PER_TILE, C_PAD), jnp.float32)
    zero_d = jnp.zeros((32, ROWS_PER_TILE, 16), jnp.float32)
    ones = jnp.ones((32, CHUNK, 16), jnp.float32)

    p, degp = _make_sc_pass(D, True, 128)(x, src2d, dst2d, zero_w, zero_d,
                                          ones)

    ws1p = jnp.pad(W_self_1, ((0, 0), (0, C_PAD - C_OUT)))
    wn1p = jnp.pad(W_neigh_1, ((0, 0), (0, C_PAD - C_OUT)))
    b1p = jnp.pad(b_1, (0, C_PAD - C_OUT)).reshape(1, C_PAD)
    b0r = b_0.reshape(1, D)

    z1, s = _tc_layer_a(x, p, degp, W_self_0, W_neigh_0, ws1p, wn1p, b0r, b1p)

    q = _make_sc_pass(C_PAD, False, 136)(z1, src2d, dst2d, zero_c)
    if isinstance(q, (list, tuple)):
        q = q[0]

    out = _tc_layer_b(q, degp, s)
    return out[:, :C_OUT]


# speedup vs baseline: 2.3334x; 2.3324x over previous
import functools

import jax
import jax.numpy as jnp
from jax import lax
from jax.experimental import pallas as pl
from jax.experimental.pallas import tpu as pltpu
from jax.experimental.pallas import tpu_sc as plsc

N = 10000
E = 320000
D = 128
C_OUT = 47
C_PAD = 64

NPAD = 10240
CHUNK = 128
NCHUNKS = 2560
EPAD = NCHUNKS * CHUNK
NW = 32
CHUNKS_PER_W = NCHUNKS // NW
ROWS_PER_TILE = NPAD // 16


def _make_sc_pass(width, with_deg, n0):
    mesh = plsc.VectorSubcoreMesh(core_axis_name="c", subcore_axis_name="s")
    nbuf = 2 if width > 64 else 4
    blk = 8
    n1 = 2 * CHUNKS_PER_W - n0
    assert n0 % blk == 0 and n1 % blk == 0
    out_type = [jax.ShapeDtypeStruct((2, NPAD, width), jnp.float32)]
    scratch = [
        pltpu.VMEM_SHARED((NPAD, width), jnp.float32),
        pltpu.VMEM((blk, CHUNK), jnp.int32),
        pltpu.VMEM((blk, CHUNK), jnp.int32),
    ] + [pltpu.VMEM((CHUNK, width), jnp.float32) for _ in range(nbuf)] \
      + [pltpu.SemaphoreType.DMA for _ in range(nbuf)]
    if with_deg:
        out_type.append(jax.ShapeDtypeStruct((2, NPAD, 16), jnp.float32))
        scratch += [
            pltpu.VMEM_SHARED((NPAD, 16), jnp.float32),
            pltpu.VMEM((CHUNK, 16), jnp.float32),
        ]

    def body(*refs):
        if with_deg:
            (table, src2d, dst2d, zero_w, zero_d, ones_hbm, outp, degp,
             agg_sp, sidx, didx, *rest) = refs
            msgs, sems = rest[:nbuf], rest[nbuf:2 * nbuf]
            deg_sp, ones_v = rest[2 * nbuf:]
        else:
            (table, src2d, dst2d, zero_w, outp,
             agg_sp, sidx, didx, *rest) = refs
            msgs, sems = rest[:nbuf], rest[nbuf:2 * nbuf]
        cid = lax.axis_index("c")
        sid = lax.axis_index("s")
        base = jnp.where(cid == 0, sid * n0, 16 * n0 + sid * n1)
        nblk_self = jnp.where(cid == 0, n0 // blk, n1 // blk)
        wslot = cid * 16 + sid
        row0 = sid * ROWS_PER_TILE
        rows = pl.ds(row0, ROWS_PER_TILE)

        with jax.named_scope("sc_zero_fill"):
            pltpu.sync_copy(zero_w.at[wslot], agg_sp.at[rows])
            if with_deg:
                pltpu.sync_copy(zero_d.at[wslot], deg_sp.at[rows])
                pltpu.sync_copy(ones_hbm.at[wslot], ones_v)
            plsc.subcore_barrier()

        with jax.named_scope("sc_edge_loop"):
            @pl.loop(0, nblk_self)
            def _(k):
                c0 = base + k * blk
                pltpu.sync_copy(src2d.at[pl.ds(c0, blk)], sidx)
                pltpu.sync_copy(dst2d.at[pl.ds(c0, blk)], didx)
                for b in range(nbuf):
                    pltpu.async_copy(table.at[sidx.at[b]], msgs[b], sems[b])
                for j in range(blk):
                    m = j % nbuf
                    pltpu.make_async_copy(table.at[sidx.at[j]], msgs[m],
                                          sems[m]).wait()
                    pltpu.sync_copy(msgs[m], agg_sp.at[didx.at[j]], add=True)
                    if with_deg:
                        pltpu.sync_copy(ones_v, deg_sp.at[didx.at[j]],
                                        add=True)
                    if j + nbuf < blk:
                        pltpu.async_copy(table.at[sidx.at[j + nbuf]],
                                         msgs[m], sems[m])

        with jax.named_scope("sc_writeback"):
            plsc.subcore_barrier()
            pltpu.sync_copy(agg_sp.at[rows], outp.at[cid, rows])
            if with_deg:
                pltpu.sync_copy(deg_sp.at[rows], degp.at[cid, rows])

    return pl.kernel(body, out_type=out_type, mesh=mesh,
                     scratch_types=scratch,
                     compiler_params=pltpu.CompilerParams(
                         use_tc_tiling_on_sc=False))


def _dot(a, b):
    return lax.dot_general(a, b, (((1,), (0,)), ((), ())),
                           precision=lax.Precision.HIGHEST,
                           preferred_element_type=jnp.float32)


def _tc_layer_a_body(x_ref, p_ref, degp_ref, ws0_ref, wn0_ref, ws1_ref,
                     wn1_ref, b0_ref, b1_ref, z1_ref, s_ref):
    deg = jnp.maximum(degp_ref[0, :, 0:1] + degp_ref[1, :, 0:1], 1.0)
    m = (p_ref[0] + p_ref[1]) / deg
    h1 = jnp.maximum(
        _dot(x_ref[...], ws0_ref[...]) + _dot(m, wn0_ref[...]) + b0_ref[...],
        0.0)
    z1_ref[...] = _dot(h1, wn1_ref[...])
    s_ref[...] = _dot(h1, ws1_ref[...]) + b1_ref[...]


def _tc_layer_b_body(q_ref, degp_ref, s_ref, out_ref):
    deg = jnp.maximum(degp_ref[0, :, 0:1] + degp_ref[1, :, 0:1], 1.0)
    out_ref[...] = s_ref[...] + (q_ref[0] + q_ref[1]) / deg


_TC_R = 1000


def _tc_layer_a(x, p, degp, ws0, wn0, ws1p, wn1p, b0, b1p):
    grid = (N // _TC_R,)
    return pl.pallas_call(
        _tc_layer_a_body,
        grid=grid,
        in_specs=[
            pl.BlockSpec((_TC_R, D), lambda i: (i, 0)),
            pl.BlockSpec((2, _TC_R, D), lambda i: (0, i, 0)),
            pl.BlockSpec((2, _TC_R, 16), lambda i: (0, i, 0)),
            pl.BlockSpec((D, D), lambda i: (0, 0)),
            pl.BlockSpec((D, D), lambda i: (0, 0)),
            pl.BlockSpec((D, C_PAD), lambda i: (0, 0)),
            pl.BlockSpec((D, C_PAD), lambda i: (0, 0)),
            pl.BlockSpec((1, D), lambda i: (0, 0)),
            pl.BlockSpec((1, C_PAD), lambda i: (0, 0)),
        ],
        out_specs=[
            pl.BlockSpec((_TC_R, C_PAD), lambda i: (i, 0)),
            pl.BlockSpec((_TC_R, C_PAD), lambda i: (i, 0)),
        ],
        out_shape=[
            jax.ShapeDtypeStruct((N, C_PAD), jnp.float32),
            jax.ShapeDtypeStruct((N, C_PAD), jnp.float32),
        ],
    )(x, p, degp, ws0, wn0, ws1p, wn1p, b0, b1p)


def _tc_layer_b(q, degp, s):
    grid = (N // _TC_R,)
    return pl.pallas_call(
        _tc_layer_b_body,
        grid=grid,
        in_specs=[
            pl.BlockSpec((2, _TC_R, C_PAD), lambda i: (0, i, 0)),
            pl.BlockSpec((2, _TC_R, 16), lambda i: (0, i, 0)),
            pl.BlockSpec((_TC_R, C_PAD), lambda i: (i, 0)),
        ],
        out_specs=pl.BlockSpec((_TC_R, C_PAD), lambda i: (i, 0)),
        out_shape=jax.ShapeDtypeStruct((N, C_PAD), jnp.float32),
    )(q, degp, s)


def kernel(x, edge_index, W_self_0, W_neigh_0, b_0, W_self_1, W_neigh_1, b_1):
    src = edge_index[0].astype(jnp.int32)
    dst = edge_index[1].astype(jnp.int32)
    npad_e = EPAD - E
    it = jnp.arange(npad_e, dtype=jnp.int32)
    src2d = jnp.concatenate([src, it % N]).reshape(NCHUNKS, CHUNK)
    dst2d = jnp.concatenate(
        [dst, N + it % (NPAD - N)]).reshape(NCHUNKS, CHUNK)

    zero_w = jnp.zeros((32, ROWS_PER_TILE, D), jnp.float32)
    zero_c = jnp.zeros((32, ROWS_PER_TILE, C_PAD), jnp.float32)
    zero_d = jnp.zeros((32, ROWS_PER_TILE, 16), jnp.float32)
    ones = jnp.ones((32, CHUNK, 16), jnp.float32)

    p, degp = _make_sc_pass(D, True, 80)(x, src2d, dst2d, zero_w, zero_d,
                                         ones)

    ws1p = jnp.pad(W_self_1, ((0, 0), (0, C_PAD - C_OUT)))
    wn1p = jnp.pad(W_neigh_1, ((0, 0), (0, C_PAD - C_OUT)))
    b1p = jnp.pad(b_1, (0, C_PAD - C_OUT)).reshape(1, C_PAD)
    b0r = b_0.reshape(1, D)

    z1, s = _tc_layer_a(x, p, degp, W_self_0, W_neigh_0, ws1p, wn1p, b0r, b1p)

    q = _make_sc_pass(C_PAD, False, 80)(z1, src2d, dst2d, zero_c)
    if isinstance(q, (list, tuple)):
        q = q[0]

    out = _tc_layer_b(q, degp, s)
    return out[:, :C_OUT]


# speedup vs baseline: 2.4037x; 1.0302x over previous
import functools

import jax
import jax.numpy as jnp
from jax import lax
from jax.experimental import pallas as pl
from jax.experimental.pallas import tpu as pltpu
from jax.experimental.pallas import tpu_sc as plsc

N = 10000
E = 320000
D = 128
C_OUT = 47
C_PAD = 64

NPAD = 10240
CHUNK = 128
NCHUNKS = 2560
EPAD = NCHUNKS * CHUNK
NW = 32
CHUNKS_PER_W = NCHUNKS // NW
ROWS_PER_TILE = NPAD // 16


def _make_sc_pass(width, with_deg, n0):
    mesh = plsc.VectorSubcoreMesh(core_axis_name="c", subcore_axis_name="s")
    nbuf = 2 if width > 64 else 4
    blk = 8
    n1 = 2 * CHUNKS_PER_W - n0
    assert n0 % blk == 0 and n1 % blk == 0
    out_type = [jax.ShapeDtypeStruct((2, NPAD, width), jnp.float32)]
    scratch = [
        pltpu.VMEM_SHARED((NPAD, width), jnp.float32),
        pltpu.VMEM((blk, CHUNK), jnp.int32),
        pltpu.VMEM((blk, CHUNK), jnp.int32),
    ] + [pltpu.VMEM((CHUNK, width), jnp.float32) for _ in range(nbuf)] \
      + [pltpu.SemaphoreType.DMA for _ in range(nbuf)]
    if with_deg:
        out_type.append(jax.ShapeDtypeStruct((2, NPAD, 16), jnp.float32))
        scratch += [
            pltpu.VMEM_SHARED((NPAD, 16), jnp.float32),
            pltpu.VMEM((CHUNK, 16), jnp.float32),
        ]

    def body(*refs):
        if with_deg:
            (table, src2d, dst2d, zero_w, zero_d, ones_hbm, outp, degp,
             agg_sp, sidx, didx, *rest) = refs
            msgs, sems = rest[:nbuf], rest[nbuf:2 * nbuf]
            deg_sp, ones_v = rest[2 * nbuf:]
        else:
            (table, src2d, dst2d, zero_w, outp,
             agg_sp, sidx, didx, *rest) = refs
            msgs, sems = rest[:nbuf], rest[nbuf:2 * nbuf]
        cid = lax.axis_index("c")
        sid = lax.axis_index("s")
        base = jnp.where(cid == 0, sid * n0, 16 * n0 + sid * n1)
        nblk_self = jnp.where(cid == 0, n0 // blk, n1 // blk)
        row0 = sid * ROWS_PER_TILE
        rows = pl.ds(row0, ROWS_PER_TILE)

        with jax.named_scope("sc_zero_fill"):
            pltpu.sync_copy(zero_w, agg_sp.at[rows])
            if with_deg:
                pltpu.sync_copy(zero_d, deg_sp.at[rows])
                pltpu.sync_copy(ones_hbm, ones_v)
            plsc.subcore_barrier()

        with jax.named_scope("sc_edge_loop"):
            @pl.loop(0, nblk_self)
            def _(k):
                c0 = base + k * blk
                pltpu.sync_copy(src2d.at[pl.ds(c0, blk)], sidx)
                pltpu.sync_copy(dst2d.at[pl.ds(c0, blk)], didx)
                for b in range(nbuf):
                    pltpu.async_copy(table.at[sidx.at[b]], msgs[b], sems[b])
                for j in range(blk):
                    m = j % nbuf
                    pltpu.make_async_copy(table.at[sidx.at[j]], msgs[m],
                                          sems[m]).wait()
                    pltpu.sync_copy(msgs[m], agg_sp.at[didx.at[j]], add=True)
                    if with_deg:
                        pltpu.sync_copy(ones_v, deg_sp.at[didx.at[j]],
                                        add=True)
                    if j + nbuf < blk:
                        pltpu.async_copy(table.at[sidx.at[j + nbuf]],
                                         msgs[m], sems[m])

        with jax.named_scope("sc_writeback"):
            plsc.subcore_barrier()
            pltpu.sync_copy(agg_sp.at[rows], outp.at[cid, rows])
            if with_deg:
                pltpu.sync_copy(deg_sp.at[rows], degp.at[cid, rows])

    return pl.kernel(body, out_type=out_type, mesh=mesh,
                     scratch_types=scratch,
                     compiler_params=pltpu.CompilerParams(
                         use_tc_tiling_on_sc=False))


def _dot(a, b):
    return lax.dot_general(a, b, (((1,), (0,)), ((), ())),
                           precision=lax.Precision.HIGHEST,
                           preferred_element_type=jnp.float32)


def _tc_self0_body(x_ref, ws0_ref, b0_ref, s0_ref):
    s0_ref[...] = _dot(x_ref[...], ws0_ref[...]) + b0_ref[...]


def _tc_layer_a_body(s0_ref, p_ref, degp_ref, wn0_ref, ws1_ref,
                     wn1_ref, b1_ref, z1_ref, s_ref):
    deg = jnp.maximum(degp_ref[0, :, 0:1] + degp_ref[1, :, 0:1], 1.0)
    m = (p_ref[0] + p_ref[1]) / deg
    h1 = jnp.maximum(s0_ref[...] + _dot(m, wn0_ref[...]), 0.0)
    z1_ref[...] = _dot(h1, wn1_ref[...])
    s_ref[...] = _dot(h1, ws1_ref[...]) + b1_ref[...]


def _tc_layer_b_body(q_ref, degp_ref, s_ref, out_ref):
    deg = jnp.maximum(degp_ref[0, :, 0:1] + degp_ref[1, :, 0:1], 1.0)
    out = s_ref[...] + (q_ref[0] + q_ref[1]) / deg
    out_ref[...] = out[:, :C_OUT]


_TC_R = 1000


def _tc_self0(x, ws0, b0):
    return pl.pallas_call(
        _tc_self0_body,
        grid=(N // _TC_R,),
        in_specs=[
            pl.BlockSpec((_TC_R, D), lambda i: (i, 0)),
            pl.BlockSpec((D, D), lambda i: (0, 0)),
            pl.BlockSpec((1, D), lambda i: (0, 0)),
        ],
        out_specs=pl.BlockSpec((_TC_R, D), lambda i: (i, 0)),
        out_shape=jax.ShapeDtypeStruct((N, D), jnp.float32),
    )(x, ws0, b0)


def _tc_layer_a(s0, p, degp, wn0, ws1p, wn1p, b1p):
    grid = (N // _TC_R,)
    return pl.pallas_call(
        _tc_layer_a_body,
        grid=grid,
        in_specs=[
            pl.BlockSpec((_TC_R, D), lambda i: (i, 0)),
            pl.BlockSpec((2, _TC_R, D), lambda i: (0, i, 0)),
            pl.BlockSpec((2, _TC_R, 16), lambda i: (0, i, 0)),
            pl.BlockSpec((D, D), lambda i: (0, 0)),
            pl.BlockSpec((D, C_PAD), lambda i: (0, 0)),
            pl.BlockSpec((D, C_PAD), lambda i: (0, 0)),
            pl.BlockSpec((1, C_PAD), lambda i: (0, 0)),
        ],
        out_specs=[
            pl.BlockSpec((_TC_R, C_PAD), lambda i: (i, 0)),
            pl.BlockSpec((_TC_R, C_PAD), lambda i: (i, 0)),
        ],
        out_shape=[
            jax.ShapeDtypeStruct((N, C_PAD), jnp.float32),
            jax.ShapeDtypeStruct((N, C_PAD), jnp.float32),
        ],
    )(s0, p, degp, wn0, ws1p, wn1p, b1p)


def _tc_layer_b(q, degp, s):
    grid = (N // _TC_R,)
    return pl.pallas_call(
        _tc_layer_b_body,
        grid=grid,
        in_specs=[
            pl.BlockSpec((2, _TC_R, C_PAD), lambda i: (0, i, 0)),
            pl.BlockSpec((2, _TC_R, 16), lambda i: (0, i, 0)),
            pl.BlockSpec((_TC_R, C_PAD), lambda i: (i, 0)),
        ],
        out_specs=pl.BlockSpec((_TC_R, C_OUT), lambda i: (i, 0)),
        out_shape=jax.ShapeDtypeStruct((N, C_OUT), jnp.float32),
    )(q, degp, s)


def kernel(x, edge_index, W_self_0, W_neigh_0, b_0, W_self_1, W_neigh_1, b_1):
    src = edge_index[0].astype(jnp.int32)
    dst = edge_index[1].astype(jnp.int32)
    npad_e = EPAD - E
    it = jnp.arange(npad_e, dtype=jnp.int32)
    src2d = jnp.concatenate([src, it % N]).reshape(NCHUNKS, CHUNK)
    dst2d = jnp.concatenate(
        [dst, N + it % (NPAD - N)]).reshape(NCHUNKS, CHUNK)

    zero_w = jnp.zeros((ROWS_PER_TILE, D), jnp.float32)
    zero_c = jnp.zeros((ROWS_PER_TILE, C_PAD), jnp.float32)
    zero_d = jnp.zeros((ROWS_PER_TILE, 16), jnp.float32)
    ones = jnp.ones((CHUNK, 16), jnp.float32)

    ws1p = jnp.pad(W_self_1, ((0, 0), (0, C_PAD - C_OUT)))
    wn1p = jnp.pad(W_neigh_1, ((0, 0), (0, C_PAD - C_OUT)))
    b1p = jnp.pad(b_1, (0, C_PAD - C_OUT)).reshape(1, C_PAD)
    b0r = b_0.reshape(1, D)

    s0 = _tc_self0(x, W_self_0, b0r)

    p, degp = _make_sc_pass(D, True, 80)(x, src2d, dst2d, zero_w, zero_d,
                                         ones)

    z1, s = _tc_layer_a(s0, p, degp, W_neigh_0, ws1p, wn1p, b1p)

    q = _make_sc_pass(C_PAD, False, 80)(z1, src2d, dst2d, zero_c)
    if isinstance(q, (list, tuple)):
        q = q[0]

    return _tc_layer_b(q, degp, s)


# speedup vs baseline: 2.4276x; 1.0099x over previous
import functools

import jax
import jax.numpy as jnp
from jax import lax
from jax.experimental import pallas as pl
from jax.experimental.pallas import tpu as pltpu
from jax.experimental.pallas import tpu_sc as plsc

N = 10000
E = 320000
D = 128
C_OUT = 47
C_PAD = 64

NPAD = 10240
CHUNK = 128
NCHUNKS = 2560
EPAD = NCHUNKS * CHUNK
NW = 32
CHUNKS_PER_W = NCHUNKS // NW
ROWS_PER_TILE = NPAD // 16


def _make_sc_pass(width, with_deg, n0):
    mesh = plsc.VectorSubcoreMesh(core_axis_name="c", subcore_axis_name="s")
    nbuf = 2 if width > 64 else 4
    blk = 8
    n1 = 2 * CHUNKS_PER_W - n0
    assert n0 % blk == 0 and n1 % blk == 0
    out_type = [jax.ShapeDtypeStruct((2, NPAD, width), jnp.float32)]
    scratch = [
        pltpu.VMEM_SHARED((NPAD, width), jnp.float32),
        pltpu.VMEM((blk, CHUNK), jnp.int32),
        pltpu.VMEM((blk, CHUNK), jnp.int32),
    ] + [pltpu.VMEM((CHUNK, width), jnp.float32) for _ in range(nbuf)] \
      + [pltpu.SemaphoreType.DMA for _ in range(2 * nbuf + 1)]
    if with_deg:
        out_type.append(jax.ShapeDtypeStruct((2, NPAD, 16), jnp.float32))
        scratch += [
            pltpu.VMEM_SHARED((NPAD, 16), jnp.float32),
            pltpu.VMEM((CHUNK, 16), jnp.float32),
        ]

    def body(*refs):
        if with_deg:
            (table, src2d, dst2d, zero_w, zero_d, ones_hbm, outp, degp,
             agg_sp, sidx, didx, *rest) = refs
            msgs, sems = rest[:nbuf], rest[nbuf:2 * nbuf]
            ssems, dsem = rest[2 * nbuf:3 * nbuf], rest[3 * nbuf]
            deg_sp, ones_v = rest[3 * nbuf + 1:]
        else:
            (table, src2d, dst2d, zero_w, outp,
             agg_sp, sidx, didx, *rest) = refs
            msgs, sems = rest[:nbuf], rest[nbuf:2 * nbuf]
            ssems, dsem = rest[2 * nbuf:3 * nbuf], rest[3 * nbuf]
        cid = lax.axis_index("c")
        sid = lax.axis_index("s")
        base = jnp.where(cid == 0, sid * n0, 16 * n0 + sid * n1)
        nblk_self = jnp.where(cid == 0, n0 // blk, n1 // blk)
        row0 = sid * ROWS_PER_TILE
        rows = pl.ds(row0, ROWS_PER_TILE)

        with jax.named_scope("sc_zero_fill"):
            pltpu.sync_copy(zero_w, agg_sp.at[rows])
            if with_deg:
                pltpu.sync_copy(zero_d, deg_sp.at[rows])
                pltpu.sync_copy(ones_hbm, ones_v)
            plsc.subcore_barrier()

        with jax.named_scope("sc_edge_loop"):
            @pl.loop(0, nblk_self)
            def _(k):
                c0 = base + k * blk
                pltpu.sync_copy(src2d.at[pl.ds(c0, blk)], sidx)
                pltpu.sync_copy(dst2d.at[pl.ds(c0, blk)], didx)
                for b in range(nbuf):
                    pltpu.async_copy(table.at[sidx.at[b]], msgs[b], sems[b])
                for j in range(blk):
                    m = j % nbuf
                    pltpu.make_async_copy(table.at[sidx.at[j]], msgs[m],
                                          sems[m]).wait()
                    pltpu.async_copy(msgs[m], agg_sp.at[didx.at[j]],
                                     ssems[m], add=True)
                    if with_deg:
                        pltpu.async_copy(ones_v, deg_sp.at[didx.at[j]],
                                         dsem, add=True)
                    if j + nbuf < blk:
                        pltpu.make_async_copy(msgs[m], agg_sp.at[didx.at[j]],
                                              ssems[m]).wait()
                        pltpu.async_copy(table.at[sidx.at[j + nbuf]],
                                         msgs[m], sems[m])
                for j in range(blk - nbuf, blk):
                    m = j % nbuf
                    pltpu.make_async_copy(msgs[m], agg_sp.at[didx.at[j]],
                                          ssems[m]).wait()
                if with_deg:
                    for j in range(blk):
                        pltpu.make_async_copy(ones_v, deg_sp.at[didx.at[j]],
                                              dsem).wait()

        with jax.named_scope("sc_writeback"):
            plsc.subcore_barrier()
            pltpu.sync_copy(agg_sp.at[rows], outp.at[cid, rows])
            if with_deg:
                pltpu.sync_copy(deg_sp.at[rows], degp.at[cid, rows])

    return pl.kernel(body, out_type=out_type, mesh=mesh,
                     scratch_types=scratch,
                     compiler_params=pltpu.CompilerParams(
                         use_tc_tiling_on_sc=False))


def _dot(a, b):
    return lax.dot_general(a, b, (((1,), (0,)), ((), ())),
                           precision=lax.Precision.HIGHEST,
                           preferred_element_type=jnp.float32)


def _tc_self0_body(x_ref, ws0_ref, b0_ref, s0_ref):
    s0_ref[...] = _dot(x_ref[...], ws0_ref[...]) + b0_ref[...]


def _tc_layer_a_body(s0_ref, p_ref, degp_ref, wn0_ref, ws1_ref,
                     wn1_ref, b1_ref, z1_ref, s_ref):
    deg = jnp.maximum(degp_ref[0, :, 0:1] + degp_ref[1, :, 0:1], 1.0)
    m = (p_ref[0] + p_ref[1]) / deg
    h1 = jnp.maximum(s0_ref[...] + _dot(m, wn0_ref[...]), 0.0)
    z1_ref[...] = _dot(h1, wn1_ref[...])
    s_ref[...] = _dot(h1, ws1_ref[...]) + b1_ref[...]


def _tc_layer_b_body(q_ref, degp_ref, s_ref, out_ref):
    deg = jnp.maximum(degp_ref[0, :, 0:1] + degp_ref[1, :, 0:1], 1.0)
    out = s_ref[...] + (q_ref[0] + q_ref[1]) / deg
    out_ref[...] = out[:, :C_OUT]


_TC_R = 1000


def _tc_self0(x, ws0, b0):
    return pl.pallas_call(
        _tc_self0_body,
        grid=(N // _TC_R,),
        in_specs=[
            pl.BlockSpec((_TC_R, D), lambda i: (i, 0)),
            pl.BlockSpec((D, D), lambda i: (0, 0)),
            pl.BlockSpec((1, D), lambda i: (0, 0)),
        ],
        out_specs=pl.BlockSpec((_TC_R, D), lambda i: (i, 0)),
        out_shape=jax.ShapeDtypeStruct((N, D), jnp.float32),
    )(x, ws0, b0)


def _tc_layer_a(s0, p, degp, wn0, ws1p, wn1p, b1p):
    grid = (N // _TC_R,)
    return pl.pallas_call(
        _tc_layer_a_body,
        grid=grid,
        in_specs=[
            pl.BlockSpec((_TC_R, D), lambda i: (i, 0)),
            pl.BlockSpec((2, _TC_R, D), lambda i: (0, i, 0)),
            pl.BlockSpec((2, _TC_R, 16), lambda i: (0, i, 0)),
            pl.BlockSpec((D, D), lambda i: (0, 0)),
            pl.BlockSpec((D, C_PAD), lambda i: (0, 0)),
            pl.BlockSpec((D, C_PAD), lambda i: (0, 0)),
            pl.BlockSpec((1, C_PAD), lambda i: (0, 0)),
        ],
        out_specs=[
            pl.BlockSpec((_TC_R, C_PAD), lambda i: (i, 0)),
            pl.BlockSpec((_TC_R, C_PAD), lambda i: (i, 0)),
        ],
        out_shape=[
            jax.ShapeDtypeStruct((N, C_PAD), jnp.float32),
            jax.ShapeDtypeStruct((N, C_PAD), jnp.float32),
        ],
    )(s0, p, degp, wn0, ws1p, wn1p, b1p)


def _tc_layer_b(q, degp, s):
    grid = (N // _TC_R,)
    return pl.pallas_call(
        _tc_layer_b_body,
        grid=grid,
        in_specs=[
            pl.BlockSpec((2, _TC_R, C_PAD), lambda i: (0, i, 0)),
            pl.BlockSpec((2, _TC_R, 16), lambda i: (0, i, 0)),
            pl.BlockSpec((_TC_R, C_PAD), lambda i: (i, 0)),
        ],
        out_specs=pl.BlockSpec((_TC_R, C_OUT), lambda i: (i, 0)),
        out_shape=jax.ShapeDtypeStruct((N, C_OUT), jnp.float32),
    )(q, degp, s)


def kernel(x, edge_index, W_self_0, W_neigh_0, b_0, W_self_1, W_neigh_1, b_1):
    src = edge_index[0].astype(jnp.int32)
    dst = edge_index[1].astype(jnp.int32)
    npad_e = EPAD - E
    it = jnp.arange(npad_e, dtype=jnp.int32)
    src2d = jnp.concatenate([src, it % N]).reshape(NCHUNKS, CHUNK)
    dst2d = jnp.concatenate(
        [dst, N + it % (NPAD - N)]).reshape(NCHUNKS, CHUNK)

    zero_w = jnp.zeros((ROWS_PER_TILE, D), jnp.float32)
    zero_c = jnp.zeros((ROWS_PER_TILE, C_PAD), jnp.float32)
    zero_d = jnp.zeros((ROWS_PER_TILE, 16), jnp.float32)
    ones = jnp.ones((CHUNK, 16), jnp.float32)

    ws1p = jnp.pad(W_self_1, ((0, 0), (0, C_PAD - C_OUT)))
    wn1p = jnp.pad(W_neigh_1, ((0, 0), (0, C_PAD - C_OUT)))
    b1p = jnp.pad(b_1, (0, C_PAD - C_OUT)).reshape(1, C_PAD)
    b0r = b_0.reshape(1, D)

    s0 = _tc_self0(x, W_self_0, b0r)

    p, degp = _make_sc_pass(D, True, 80)(x, src2d, dst2d, zero_w, zero_d,
                                         ones)

    z1, s = _tc_layer_a(s0, p, degp, W_neigh_0, ws1p, wn1p, b1p)

    q = _make_sc_pass(C_PAD, False, 80)(z1, src2d, dst2d, zero_c)
    if isinstance(q, (list, tuple)):
        q = q[0]

    return _tc_layer_b(q, degp, s)
